# baseline (speedup 1.0000x reference)
import functools

import jax
import jax.numpy as jnp
from jax.experimental import pallas as pl
from jax.experimental.pallas import tpu as pltpu

F32 = jnp.float32
BF16 = jnp.bfloat16


def _split_hi_lo(x):
    hi = x.astype(BF16)
    lo = (x - hi.astype(F32)).astype(BF16)
    return hi, lo


def _mm(a_bf16, x, dims):
    hi, lo = _split_hi_lo(x)
    return (jax.lax.dot_general(a_bf16, hi, dims, preferred_element_type=F32)
            + jax.lax.dot_general(a_bf16, lo, dims, preferred_element_type=F32))


_NN = (((1,), (0,)), ((), ()))
_TN = (((0,), (0,)), ((), ()))


def _xw_body(x_ref, w_ref, o_ref):
    o_ref[...] = jnp.dot(x_ref[...], w_ref[...], preferred_element_type=F32)


def _xw(x, w, bm):
    n, k = x.shape
    h = w.shape[1]
    return pl.pallas_call(
        _xw_body,
        grid=(n // bm,),
        in_specs=[
            pl.BlockSpec((bm, k), lambda i: (i, 0)),
            pl.BlockSpec((k, h), lambda i: (0, 0)),
        ],
        out_specs=pl.BlockSpec((bm, h), lambda i: (i, 0)),
        out_shape=jax.ShapeDtypeStruct((n, h), F32),
    )(x, w)


def _pack_matrix(t_bf16, b):
    w = b // 8
    wi = jax.lax.broadcasted_iota(jnp.int32, (w, b), 0)
    ri = jax.lax.broadcasted_iota(jnp.int32, (w, b), 1)
    weight = jnp.exp2((ri // w).astype(F32))
    p = (ri % w == wi).astype(F32) * weight
    return jnp.dot(p.astype(BF16), t_bf16, preferred_element_type=F32)


def _unpack_matrix(words_i32, b):
    w = b // 8
    rep = jnp.concatenate([words_i32] * 8, axis=0)
    shift = jax.lax.broadcasted_iota(jnp.int32, (b, 1), 0) // w
    return ((rep >> shift) & 1).astype(BF16)


def _h_body(adj_ref, xw1_ref, xw2_ref, h1_ref, h2_ref, pk_ref, acc1, *, ni):
    i = pl.program_id(0)
    t = adj_ref[...].astype(BF16)
    pk_ref[...] = _pack_matrix(t, adj_ref.shape[0]).astype(jnp.int32)[None]
    h2_ref[...] = jnp.maximum(_mm(t, xw2_ref[...], _NN), 0.0)
    c1 = _mm(t, xw1_ref[...], _TN)

    @pl.when(i == 0)
    def _():
        acc1[...] = c1

    @pl.when(i != 0)
    def _():
        acc1[...] += c1

    @pl.when(i == ni - 1)
    def _():
        h1_ref[...] = jnp.maximum(acc1[...], 0.0)


def _propagate_in(adj, xw1, xw2, b):
    n1, n2 = adj.shape
    h = xw1.shape[1]
    ni = n1 // b
    return pl.pallas_call(
        functools.partial(_h_body, ni=ni),
        grid=(ni,),
        in_specs=[
            pl.BlockSpec((b, n2), lambda i: (i, 0)),
            pl.BlockSpec((b, h), lambda i: (i, 0)),
            pl.BlockSpec((n2, h), lambda i: (0, 0)),
        ],
        out_specs=[
            pl.BlockSpec((n2, h), lambda i: (0, 0)),
            pl.BlockSpec((b, h), lambda i: (i, 0)),
            pl.BlockSpec((1, b // 8, n2), lambda i: (i, 0, 0)),
        ],
        out_shape=[
            jax.ShapeDtypeStruct((n2, h), F32),
            jax.ShapeDtypeStruct((n1, h), F32),
            jax.ShapeDtypeStruct((n1 // b, b // 8, n2), jnp.int32),
        ],
        scratch_shapes=[pltpu.VMEM((n2, h), F32)],
    )(adj, xw1, xw2)


def _z_body(pk_ref, h1_ref, h2_ref, wm1_ref, wl1_ref, wm2_ref, wl2_ref,
            n1_ref, n2_ref, z1_ref, z2_ref, acc2, *, ni, b):
    i = pl.program_id(0)
    t = _unpack_matrix(pk_ref[0], b)
    ah1 = _mm(t, h1_ref[...], _NN)
    mean1 = jnp.dot(ah1, wm1_ref[...], preferred_element_type=F32)
    logstd1 = jnp.dot(ah1, wl1_ref[...], preferred_element_type=F32)
    z1_ref[...] = n1_ref[...] * jnp.exp(logstd1) + mean1

    c2 = _mm(t, h2_ref[...], _TN)

    @pl.when(i == 0)
    def _():
        acc2[...] = c2

    @pl.when(i != 0)
    def _():
        acc2[...] += c2

    @pl.when(i == ni - 1)
    def _():
        ah2 = acc2[...]
        mean2 = jnp.dot(ah2, wm2_ref[...], preferred_element_type=F32)
        logstd2 = jnp.dot(ah2, wl2_ref[...], preferred_element_type=F32)
        z2_ref[...] = n2_ref[...] * jnp.exp(logstd2) + mean2


def _propagate_out(pk, n1, h1, h2, wm1, wl1, wm2, wl2, noise1, noise2, b):
    n2 = pk.shape[2]
    h = h1.shape[1]
    h2dim = wm1.shape[1]
    ni = n1 // b
    full = lambda a: pl.BlockSpec(a.shape, lambda i: tuple(0 for _ in a.shape))
    return pl.pallas_call(
        functools.partial(_z_body, ni=ni, b=b),
        grid=(ni,),
        in_specs=[
            pl.BlockSpec((1, b // 8, n2), lambda i: (i, 0, 0)),
            full(h1),
            pl.BlockSpec((b, h), lambda i: (i, 0)),
            full(wm1), full(wl1), full(wm2), full(wl2),
            pl.BlockSpec((b, h2dim), lambda i: (i, 0)),
            full(noise2),
        ],
        out_specs=[
            pl.BlockSpec((b, h2dim), lambda i: (i, 0)),
            pl.BlockSpec((n2, h2dim), lambda i: (0, 0)),
        ],
        out_shape=[
            jax.ShapeDtypeStruct((n1, h2dim), F32),
            jax.ShapeDtypeStruct((n2, h2dim), F32),
        ],
        scratch_shapes=[pltpu.VMEM((n2, h), F32)],
    )(pk, h1, h2, wm1, wl1, wm2, wl2, noise1, noise2)


def _dec_body(z1_ref, z2_ref, a_ref):
    logits = jax.lax.dot_general(z1_ref[...], z2_ref[...],
                                 (((1,), (1,)), ((), ())),
                                 preferred_element_type=F32)
    a_ref[...] = jax.nn.sigmoid(logits)


def _decode(z1, z2, bm):
    n1, h2dim = z1.shape
    n2 = z2.shape[0]
    return pl.pallas_call(
        _dec_body,
        grid=(n1 // bm,),
        in_specs=[
            pl.BlockSpec((bm, h2dim), lambda i: (i, 0)),
            pl.BlockSpec((n2, h2dim), lambda i: (0, 0)),
        ],
        out_specs=pl.BlockSpec((bm, n2), lambda i: (i, 0)),
        out_shape=jax.ShapeDtypeStruct((n1, n2), F32),
    )(z1, z2)


def kernel(X1, X2, adj, W_base1, W_mean1, W_logstd1, W_base2, W_mean2,
           W_logstd2, noise1, noise2):
    n1, n2 = adj.shape
    bm = max(n1 // 50, 1)

    xw1 = _xw(X1, W_base1, bm)
    xw2 = _xw(X2, W_base2, bm)
    h1, h2, pk = _propagate_in(adj, xw1, xw2, bm)
    z1, z2 = _propagate_out(pk, n1, h1, h2, W_mean1, W_logstd1, W_mean2,
                            W_logstd2, noise1, noise2, bm)
    a_pred = _decode(z1, z2, bm)
    return (a_pred, z1, z2)


# speedup vs baseline: 1.2644x; 1.2644x over previous
import functools

import jax
import jax.numpy as jnp
from jax.experimental import pallas as pl
from jax.experimental.pallas import tpu as pltpu

F32 = jnp.float32
BF16 = jnp.bfloat16


def _split_hi_lo(x):
    hi = x.astype(BF16)
    lo = (x - hi.astype(F32)).astype(BF16)
    return hi, lo


def _mm(a_bf16, x, dims):
    hi, lo = _split_hi_lo(x)
    return (jax.lax.dot_general(a_bf16, hi, dims, preferred_element_type=F32)
            + jax.lax.dot_general(a_bf16, lo, dims, preferred_element_type=F32))


_NN = (((1,), (0,)), ((), ()))
_TN = (((0,), (0,)), ((), ()))


def _xw_body(x_ref, w_ref, o_ref):
    o_ref[...] = jnp.dot(x_ref[...], w_ref[...], preferred_element_type=F32)


def _xw(x, w, bm):
    n, k = x.shape
    h = w.shape[1]
    return pl.pallas_call(
        _xw_body,
        grid=(n // bm,),
        in_specs=[
            pl.BlockSpec((bm, k), lambda i: (i, 0)),
            pl.BlockSpec((k, h), lambda i: (0, 0)),
        ],
        out_specs=pl.BlockSpec((bm, h), lambda i: (i, 0)),
        out_shape=jax.ShapeDtypeStruct((n, h), F32),
    )(x, w)


def _h_body(adj_ref, xw1_ref, xw2_ref, h1_ref, h2_ref, acc1, *, ni):
    i = pl.program_id(0)
    t = adj_ref[...]
    h2_ref[...] = jnp.maximum(
        jax.lax.dot_general(t, xw2_ref[...], _NN, preferred_element_type=F32),
        0.0)
    c1 = jax.lax.dot_general(t, xw1_ref[...], _TN, preferred_element_type=F32)

    @pl.when(i == 0)
    def _():
        acc1[...] = c1

    @pl.when(i != 0)
    def _():
        acc1[...] += c1

    @pl.when(i == ni - 1)
    def _():
        h1_ref[...] = jnp.maximum(acc1[...], 0.0)


def _propagate_in(adj, xw1, xw2, b):
    n1, n2 = adj.shape
    h = xw1.shape[1]
    ni = n1 // b
    return pl.pallas_call(
        functools.partial(_h_body, ni=ni),
        grid=(ni,),
        in_specs=[
            pl.BlockSpec((b, n2), lambda i: (i, 0)),
            pl.BlockSpec((b, h), lambda i: (i, 0)),
            pl.BlockSpec((n2, h), lambda i: (0, 0)),
        ],
        out_specs=[
            pl.BlockSpec((n2, h), lambda i: (0, 0)),
            pl.BlockSpec((b, h), lambda i: (i, 0)),
        ],
        out_shape=[
            jax.ShapeDtypeStruct((n2, h), F32),
            jax.ShapeDtypeStruct((n1, h), F32),
        ],
        scratch_shapes=[pltpu.VMEM((n2, h), F32)],
    )(adj, xw1, xw2)


def _z_body(adj_ref, h1_ref, h2_ref, wm1_ref, wl1_ref, wm2_ref, wl2_ref,
            n1_ref, n2_ref, z1_ref, z2_ref, acc2, *, ni):
    i = pl.program_id(0)
    t = adj_ref[...]
    ah1 = jax.lax.dot_general(t, h1_ref[...], _NN, preferred_element_type=F32)
    mean1 = jnp.dot(ah1, wm1_ref[...], preferred_element_type=F32)
    logstd1 = jnp.dot(ah1, wl1_ref[...], preferred_element_type=F32)
    z1_ref[...] = n1_ref[...] * jnp.exp(logstd1) + mean1

    c2 = jax.lax.dot_general(t, h2_ref[...], _TN, preferred_element_type=F32)

    @pl.when(i == 0)
    def _():
        acc2[...] = c2

    @pl.when(i != 0)
    def _():
        acc2[...] += c2

    @pl.when(i == ni - 1)
    def _():
        ah2 = acc2[...]
        mean2 = jnp.dot(ah2, wm2_ref[...], preferred_element_type=F32)
        logstd2 = jnp.dot(ah2, wl2_ref[...], preferred_element_type=F32)
        z2_ref[...] = n2_ref[...] * jnp.exp(logstd2) + mean2


def _propagate_out(adj, h1, h2, wm1, wl1, wm2, wl2, noise1, noise2, b):
    n1, n2 = adj.shape
    h = h1.shape[1]
    h2dim = wm1.shape[1]
    ni = n1 // b
    full = lambda a: pl.BlockSpec(a.shape, lambda i: tuple(0 for _ in a.shape))
    return pl.pallas_call(
        functools.partial(_z_body, ni=ni),
        grid=(ni,),
        in_specs=[
            pl.BlockSpec((b, n2), lambda i: (i, 0)),
            full(h1),
            pl.BlockSpec((b, h), lambda i: (i, 0)),
            full(wm1), full(wl1), full(wm2), full(wl2),
            pl.BlockSpec((b, h2dim), lambda i: (i, 0)),
            full(noise2),
        ],
        out_specs=[
            pl.BlockSpec((b, h2dim), lambda i: (i, 0)),
            pl.BlockSpec((n2, h2dim), lambda i: (0, 0)),
        ],
        out_shape=[
            jax.ShapeDtypeStruct((n1, h2dim), F32),
            jax.ShapeDtypeStruct((n2, h2dim), F32),
        ],
        scratch_shapes=[pltpu.VMEM((n2, h), F32)],
    )(adj, h1, h2, wm1, wl1, wm2, wl2, noise1, noise2)


def _dec_body(z1_ref, z2_ref, a_ref):
    logits = jax.lax.dot_general(z1_ref[...], z2_ref[...],
                                 (((1,), (1,)), ((), ())),
                                 preferred_element_type=F32)
    a_ref[...] = jax.nn.sigmoid(logits)


def _decode(z1, z2, bm):
    n1, h2dim = z1.shape
    n2 = z2.shape[0]
    return pl.pallas_call(
        _dec_body,
        grid=(n1 // bm,),
        in_specs=[
            pl.BlockSpec((bm, h2dim), lambda i: (i, 0)),
            pl.BlockSpec((n2, h2dim), lambda i: (0, 0)),
        ],
        out_specs=pl.BlockSpec((bm, n2), lambda i: (i, 0)),
        out_shape=jax.ShapeDtypeStruct((n1, n2), F32),
    )(z1, z2)


def kernel(X1, X2, adj, W_base1, W_mean1, W_logstd1, W_base2, W_mean2,
           W_logstd2, noise1, noise2):
    n1, n2 = adj.shape
    bm = max(n1 // 50, 1)

    xw1 = _xw(X1, W_base1, bm)
    xw2 = _xw(X2, W_base2, bm)
    h1, h2 = _propagate_in(adj, xw1, xw2, bm)
    z1, z2 = _propagate_out(adj, h1, h2, W_mean1, W_logstd1, W_mean2,
                            W_logstd2, noise1, noise2, bm)
    a_pred = _decode(z1, z2, bm)
    return (a_pred, z1, z2)


# speedup vs baseline: 1.5142x; 1.1975x over previous
import functools

import jax
import jax.numpy as jnp
from jax.experimental import pallas as pl
from jax.experimental.pallas import tpu as pltpu

F32 = jnp.float32
BF16 = jnp.bfloat16

_NN = (((1,), (0,)), ((), ()))
_TT = (((0,), (0,)), ((), ()))


def _hi_lo_concat(x, axis=1):
    hi = x.astype(BF16)
    lo = (x - hi.astype(F32)).astype(BF16)
    return jnp.concatenate([hi, lo], axis=axis)


def _sum_halves(y, axis=1):
    w = y.shape[axis] // 2
    if axis == 0:
        return y[:w] + y[w:]
    return y[:, :w] + y[:, w:]


def _xw_body(x_ref, w_ref, o_ref):
    xw = jnp.dot(x_ref[...], w_ref[...], preferred_element_type=F32)
    o_ref[...] = _hi_lo_concat(xw)


def _xw(x, w, bm):
    n, k = x.shape
    h = w.shape[1]
    return pl.pallas_call(
        _xw_body,
        grid=(n // bm,),
        in_specs=[
            pl.BlockSpec((bm, k), lambda i: (i, 0)),
            pl.BlockSpec((k, h), lambda i: (0, 0)),
        ],
        out_specs=pl.BlockSpec((bm, 2 * h), lambda i: (i, 0)),
        out_shape=jax.ShapeDtypeStruct((n, 2 * h), BF16),
    )(x, w)


def _h_body(adj_ref, xw1_ref, xw2_ref, h1_ref, h2_ref, acc1, *, ni):
    i = pl.program_id(0)
    t = adj_ref[...].astype(BF16)
    h2 = jnp.maximum(
        _sum_halves(jax.lax.dot_general(t, xw2_ref[...], _NN,
                                        preferred_element_type=F32)), 0.0)
    h2_ref[...] = _hi_lo_concat(h2)
    c1t = jax.lax.dot_general(xw1_ref[...], t, _TT,
                              preferred_element_type=F32)

    @pl.when(i == 0)
    def _():
        acc1[...] = c1t

    @pl.when(i != 0)
    def _():
        acc1[...] += c1t

    @pl.when(i == ni - 1)
    def _():
        h1t = jnp.maximum(_sum_halves(acc1[...], axis=0), 0.0)
        h1_ref[...] = _hi_lo_concat(h1t.T)


def _propagate_in(adj, xw1, xw2, b):
    n1, n2 = adj.shape
    h2w = xw1.shape[1]
    ni = n1 // b
    return pl.pallas_call(
        functools.partial(_h_body, ni=ni),
        grid=(ni,),
        in_specs=[
            pl.BlockSpec((b, n2), lambda i: (i, 0)),
            pl.BlockSpec((b, h2w), lambda i: (i, 0)),
            pl.BlockSpec((n2, h2w), lambda i: (0, 0)),
        ],
        out_specs=[
            pl.BlockSpec((n2, h2w), lambda i: (0, 0)),
            pl.BlockSpec((b, h2w), lambda i: (i, 0)),
        ],
        out_shape=[
            jax.ShapeDtypeStruct((n2, h2w), BF16),
            jax.ShapeDtypeStruct((n1, h2w), BF16),
        ],
        scratch_shapes=[pltpu.VMEM((h2w, n2), F32)],
    )(adj, xw1, xw2)


def _z_body(adj_ref, h1_ref, h2_ref, wm1_ref, wl1_ref, wm2_ref, wl2_ref,
            n1_ref, n2_ref, z1_ref, z2_ref, acc2, *, ni):
    i = pl.program_id(0)
    t = adj_ref[...].astype(BF16)
    ah1 = _sum_halves(jax.lax.dot_general(t, h1_ref[...], _NN,
                                          preferred_element_type=F32))
    mean1 = jnp.dot(ah1, wm1_ref[...], preferred_element_type=F32)
    logstd1 = jnp.dot(ah1, wl1_ref[...], preferred_element_type=F32)
    z1_ref[...] = n1_ref[...] * jnp.exp(logstd1) + mean1

    c2t = jax.lax.dot_general(h2_ref[...], t, _TT,
                              preferred_element_type=F32)

    @pl.when(i == 0)
    def _():
        acc2[...] = c2t

    @pl.when(i != 0)
    def _():
        acc2[...] += c2t

    @pl.when(i == ni - 1)
    def _():
        ah2t = _sum_halves(acc2[...], axis=0)
        mean2t = jax.lax.dot_general(wm2_ref[...], ah2t, _TT,
                                     preferred_element_type=F32)
        logstd2t = jax.lax.dot_general(wl2_ref[...], ah2t, _TT,
                                       preferred_element_type=F32)
        z2_ref[...] = (n2_ref[...].T * jnp.exp(logstd2t) + mean2t).T


def _propagate_out(adj, h1, h2, wm1, wl1, wm2, wl2, noise1, noise2, b):
    n1, n2 = adj.shape
    h2w = h1.shape[1]
    hz = wm1.shape[1]
    ni = n1 // b
    full = lambda a: pl.BlockSpec(a.shape, lambda i: tuple(0 for _ in a.shape))
    return pl.pallas_call(
        functools.partial(_z_body, ni=ni),
        grid=(ni,),
        in_specs=[
            pl.BlockSpec((b, n2), lambda i: (i, 0)),
            full(h1),
            pl.BlockSpec((b, h2w), lambda i: (i, 0)),
            full(wm1), full(wl1), full(wm2), full(wl2),
            pl.BlockSpec((b, hz), lambda i: (i, 0)),
            full(noise2),
        ],
        out_specs=[
            pl.BlockSpec((b, hz), lambda i: (i, 0)),
            pl.BlockSpec((n2, hz), lambda i: (0, 0)),
        ],
        out_shape=[
            jax.ShapeDtypeStruct((n1, hz), F32),
            jax.ShapeDtypeStruct((n2, hz), F32),
        ],
        scratch_shapes=[pltpu.VMEM((h2w, n2), F32)],
    )(adj, h1, h2, wm1, wl1, wm2, wl2, noise1, noise2)


def _dec_body(z1_ref, z2_ref, a_ref):
    logits = jax.lax.dot_general(z1_ref[...], z2_ref[...],
                                 (((1,), (1,)), ((), ())),
                                 preferred_element_type=F32)
    a_ref[...] = jax.nn.sigmoid(logits)


def _decode(z1, z2, bm):
    n1, hz = z1.shape
    n2 = z2.shape[0]
    return pl.pallas_call(
        _dec_body,
        grid=(n1 // bm,),
        in_specs=[
            pl.BlockSpec((bm, hz), lambda i: (i, 0)),
            pl.BlockSpec((n2, hz), lambda i: (0, 0)),
        ],
        out_specs=pl.BlockSpec((bm, n2), lambda i: (i, 0)),
        out_shape=jax.ShapeDtypeStruct((n1, n2), F32),
    )(z1, z2)


def kernel(X1, X2, adj, W_base1, W_mean1, W_logstd1, W_base2, W_mean2,
           W_logstd2, noise1, noise2):
    n1, n2 = adj.shape
    bm = max(n1 // 50, 1)

    xw1 = _xw(X1, W_base1, bm)
    xw2 = _xw(X2, W_base2, bm)
    h1, h2 = _propagate_in(adj, xw1, xw2, bm)
    z1, z2 = _propagate_out(adj, h1, h2, W_mean1, W_logstd1, W_mean2,
                            W_logstd2, noise1, noise2, bm)
    a_pred = _decode(z1, z2, bm)
    return (a_pred, z1, z2)


# speedup vs baseline: 1.5692x; 1.0364x over previous
import functools

import jax
import jax.numpy as jnp
from jax.experimental import pallas as pl
from jax.experimental.pallas import tpu as pltpu

F32 = jnp.float32
BF16 = jnp.bfloat16

_NN = (((1,), (0,)), ((), ()))
_TT = (((0,), (0,)), ((), ()))


def _hi_lo_concat(x, axis=1):
    hi = x.astype(BF16)
    lo = (x - hi.astype(F32)).astype(BF16)
    return jnp.concatenate([hi, lo], axis=axis)


def _sum_halves(y, axis=1):
    w = y.shape[axis] // 2
    if axis == 0:
        return y[:w] + y[w:]
    return y[:, :w] + y[:, w:]


def _xw_body(x_ref, w_ref, o_ref):
    xw = jnp.dot(x_ref[...], w_ref[...], preferred_element_type=F32)
    o_ref[...] = _hi_lo_concat(xw)


def _xw(x, w, bm):
    n, k = x.shape
    h = w.shape[1]
    return pl.pallas_call(
        _xw_body,
        grid=(n // bm,),
        in_specs=[
            pl.BlockSpec((bm, k), lambda i: (i, 0)),
            pl.BlockSpec((k, h), lambda i: (0, 0)),
        ],
        out_specs=pl.BlockSpec((bm, 2 * h), lambda i: (i, 0)),
        out_shape=jax.ShapeDtypeStruct((n, 2 * h), BF16),
    )(x, w)


def _h_body(adj_ref, xw1_ref, xw2_ref, h1_ref, h2_ref, acc1, *, ni):
    i = pl.program_id(0)
    t = adj_ref[...].astype(BF16)
    h2 = jnp.maximum(
        _sum_halves(jax.lax.dot_general(t, xw2_ref[...], _NN,
                                        preferred_element_type=F32)), 0.0)
    h2_ref[...] = _hi_lo_concat(h2)
    c1t = jax.lax.dot_general(xw1_ref[...], t, _TT,
                              preferred_element_type=F32)

    @pl.when(i == 0)
    def _():
        acc1[...] = c1t

    @pl.when(i != 0)
    def _():
        acc1[...] += c1t

    @pl.when(i == ni - 1)
    def _():
        h1t = jnp.maximum(_sum_halves(acc1[...], axis=0), 0.0)
        h1_ref[...] = _hi_lo_concat(h1t.T)


def _propagate_in(adj, xw1, xw2, b):
    n1, n2 = adj.shape
    h2w = xw1.shape[1]
    ni = n1 // b
    return pl.pallas_call(
        functools.partial(_h_body, ni=ni),
        grid=(ni,),
        in_specs=[
            pl.BlockSpec((b, n2), lambda i: (i, 0)),
            pl.BlockSpec((b, h2w), lambda i: (i, 0)),
            pl.BlockSpec((n2, h2w), lambda i: (0, 0)),
        ],
        out_specs=[
            pl.BlockSpec((n2, h2w), lambda i: (0, 0)),
            pl.BlockSpec((b, h2w), lambda i: (i, 0)),
        ],
        out_shape=[
            jax.ShapeDtypeStruct((n2, h2w), BF16),
            jax.ShapeDtypeStruct((n1, h2w), BF16),
        ],
        scratch_shapes=[pltpu.VMEM((h2w, n2), F32)],
    )(adj, xw1, xw2)


def _z_body(adj_ref, h1_ref, h2_ref, wm1_ref, wl1_ref, wm2_ref, wl2_ref,
            n1_ref, n2_ref, z1_ref, z2_ref, acc2, *, ni):
    i = pl.program_id(0)
    t = adj_ref[...].astype(BF16)
    ah1 = _sum_halves(jax.lax.dot_general(t, h1_ref[...], _NN,
                                          preferred_element_type=F32))
    mean1 = jnp.dot(ah1, wm1_ref[...], preferred_element_type=F32)
    logstd1 = jnp.dot(ah1, wl1_ref[...], preferred_element_type=F32)
    z1_ref[...] = n1_ref[...] * jnp.exp(logstd1) + mean1

    c2t = jax.lax.dot_general(h2_ref[...], t, _TT,
                              preferred_element_type=F32)

    @pl.when(i == 0)
    def _():
        acc2[...] = c2t

    @pl.when(i != 0)
    def _():
        acc2[...] += c2t

    @pl.when(i == ni - 1)
    def _():
        ah2t = _sum_halves(acc2[...], axis=0)
        mean2t = jax.lax.dot_general(wm2_ref[...], ah2t, _TT,
                                     preferred_element_type=F32)
        logstd2t = jax.lax.dot_general(wl2_ref[...], ah2t, _TT,
                                       preferred_element_type=F32)
        z2_ref[...] = (n2_ref[...].T * jnp.exp(logstd2t) + mean2t).T


def _propagate_out(adj, h1, h2, wm1, wl1, wm2, wl2, noise1, noise2, b):
    n1, n2 = adj.shape
    h2w = h1.shape[1]
    hz = wm1.shape[1]
    ni = n1 // b
    full = lambda a: pl.BlockSpec(a.shape, lambda i: tuple(0 for _ in a.shape))
    return pl.pallas_call(
        functools.partial(_z_body, ni=ni),
        grid=(ni,),
        in_specs=[
            pl.BlockSpec((b, n2), lambda i: (i, 0)),
            full(h1),
            pl.BlockSpec((b, h2w), lambda i: (i, 0)),
            full(wm1), full(wl1), full(wm2), full(wl2),
            pl.BlockSpec((b, hz), lambda i: (i, 0)),
            full(noise2),
        ],
        out_specs=[
            pl.BlockSpec((b, hz), lambda i: (i, 0)),
            pl.BlockSpec((n2, hz), lambda i: (0, 0)),
        ],
        out_shape=[
            jax.ShapeDtypeStruct((n1, hz), F32),
            jax.ShapeDtypeStruct((n2, hz), F32),
        ],
        scratch_shapes=[pltpu.VMEM((h2w, n2), F32)],
    )(adj, h1, h2, wm1, wl1, wm2, wl2, noise1, noise2)


def _dec_body(z1_ref, z2_ref, a_ref):
    logits = jax.lax.dot_general(z1_ref[...], z2_ref[...],
                                 (((1,), (1,)), ((), ())),
                                 preferred_element_type=F32)
    a_ref[...] = jax.nn.sigmoid(logits)


def _decode(z1, z2, bm):
    n1, hz = z1.shape
    n2 = z2.shape[0]
    return pl.pallas_call(
        _dec_body,
        grid=(n1 // bm,),
        in_specs=[
            pl.BlockSpec((bm, hz), lambda i: (i, 0)),
            pl.BlockSpec((n2, hz), lambda i: (0, 0)),
        ],
        out_specs=pl.BlockSpec((bm, n2), lambda i: (i, 0)),
        out_shape=jax.ShapeDtypeStruct((n1, n2), F32),
    )(z1, z2)


def kernel(X1, X2, adj, W_base1, W_mean1, W_logstd1, W_base2, W_mean2,
           W_logstd2, noise1, noise2):
    n1, n2 = adj.shape
    bm = max(n1 // 25, 1)

    xw1 = _xw(X1, W_base1, bm)
    xw2 = _xw(X2, W_base2, bm)
    h1, h2 = _propagate_in(adj, xw1, xw2, bm)
    z1, z2 = _propagate_out(adj, h1, h2, W_mean1, W_logstd1, W_mean2,
                            W_logstd2, noise1, noise2, bm)
    a_pred = _decode(z1, z2, bm)
    return (a_pred, z1, z2)
